# Initial kernel scaffold; baseline (speedup 1.0000x reference)
import jax
import jax.numpy as jnp
from jax.experimental import pallas as pl


def kernel(x, edge_index, batch, W0, att_src0, att_dst0, b0, W1, att_src1, att_dst1, b1, W2, att_src2, att_dst2, b2, fc1_w, fc1_b, fc2_w, fc2_b):
    raise NotImplementedError("write your pallas kernel here")


# speedup vs baseline: 25.4077x; 25.4077x over previous
import functools
import jax
import jax.numpy as jnp
from jax import lax
from jax.experimental import pallas as pl
from jax.experimental.pallas import tpu as pltpu
from jax.experimental.pallas import tpu_sc as plsc

N = 10000
D = 128
G = 128
FC = 256
NC = 2
NS = 16
NW = NC * NS
K = 128
ROWS_PAD = 10240
TILE_ROWS = ROWS_PAD // NS
NBLK = 10
RBLK = N // NBLK


def _num_chunks(n_edges_total):
    per_worker = -(-n_edges_total // (NW * K)) * K
    return per_worker // K, per_worker


def _sc_edge_body(h_hbm, as_hbm, ad_hbm, src_hbm, dst_hbm,
                  outp_hbm, denp_hbm,
                  asv, adv, srcv, dstv, pv, rows, sem, n_chunks, per_worker):
    cid = lax.axis_index("c")
    sid = lax.axis_index("s")
    wid = sid * NC + cid
    tbase = sid * TILE_ROWS

    zv = jnp.zeros((16,), jnp.float32)

    def zrow(i, _):
        for f in range(8):
            rows[i, pl.ds(f * 16, 16)] = zv
        return 0

    lax.fori_loop(0, K, zrow, 0)

    def zp(i, _):
        pv[pl.ds(i * 16, 16)] = zv
        return 0

    lax.fori_loop(0, K // 16, zp, 0)

    def zout(r, _):
        pltpu.sync_copy(rows, outp_hbm.at[pl.ds(tbase + r * K, K)])
        pltpu.sync_copy(pv, denp_hbm.at[pl.ds(tbase + r * K, K)])
        return 0

    lax.fori_loop(0, TILE_ROWS // K, zout, 0)
    plsc.subcore_barrier()

    pltpu.sync_copy(as_hbm, asv)
    pltpu.sync_copy(ad_hbm, adv)

    def chunk(c, _):
        base = wid * per_worker + c * K
        pltpu.sync_copy(src_hbm.at[pl.ds(base, K)], srcv)
        pltpu.sync_copy(dst_hbm.at[pl.ds(base, K)], dstv)
        for g in range(K // 16):
            si = srcv[pl.ds(g * 16, 16)]
            di = dstv[pl.ds(g * 16, 16)]
            e = plsc.load_gather(asv, [si]) + plsc.load_gather(adv, [di])
            e = jnp.where(e > 0.0, e, 0.2 * e)
            pv[pl.ds(g * 16, 16)] = jnp.exp(e)
        pltpu.sync_copy(pv, denp_hbm.at[dstv], add=True)
        pltpu.async_copy(h_hbm.at[srcv], rows, sem).wait()

        def scale(g, _):
            p16 = pv[pl.ds(g * 16, 16)]
            base16 = g * 16
            for j in range(16):
                a = p16[j]
                for f in range(8):
                    rows[base16 + j, pl.ds(f * 16, 16)] = (
                        rows[base16 + j, pl.ds(f * 16, 16)] * a)
            return 0

        lax.fori_loop(0, K // 16, scale, 0)
        pltpu.sync_copy(rows, outp_hbm.at[dstv], add=True)
        return 0

    lax.fori_loop(0, n_chunks, chunk, 0)
    plsc.subcore_barrier()


def _make_sc_edge(n_chunks, per_worker):
    mesh = plsc.VectorSubcoreMesh(core_axis_name="c", subcore_axis_name="s",
                                  num_cores=NC, num_subcores=NS)

    def body(h_hbm, as_hbm, ad_hbm, src_hbm, dst_hbm, outp_hbm, denp_hbm,
             acc, den, asv, adv, srcv, dstv, pv, rows, sem):
        cid = lax.axis_index("c")
        sid = lax.axis_index("s")
        _sc_edge_body(h_hbm, as_hbm, ad_hbm, src_hbm, dst_hbm,
                      acc, den, asv, adv, srcv, dstv, pv, rows, sem,
                      n_chunks, per_worker)
        tbase = sid * TILE_ROWS
        pltpu.sync_copy(acc.at[pl.ds(tbase, TILE_ROWS)],
                        outp_hbm.at[cid, pl.ds(tbase, TILE_ROWS)])
        pltpu.sync_copy(den.at[pl.ds(tbase, TILE_ROWS)],
                        denp_hbm.at[cid, pl.ds(tbase, TILE_ROWS)])

    return pl.kernel(
        body,
        out_type=[
            jax.ShapeDtypeStruct((NC, ROWS_PAD, D), jnp.float32),
            jax.ShapeDtypeStruct((NC, ROWS_PAD), jnp.float32),
        ],
        mesh=mesh,
        compiler_params=pltpu.CompilerParams(needs_layout_passes=False),
        scratch_types=[
            pltpu.VMEM_SHARED((ROWS_PAD, D), jnp.float32),
            pltpu.VMEM_SHARED((ROWS_PAD,), jnp.float32),
            pltpu.VMEM((N,), jnp.float32),
            pltpu.VMEM((N,), jnp.float32),
            pltpu.VMEM((K,), jnp.int32),
            pltpu.VMEM((K,), jnp.int32),
            pltpu.VMEM((K,), jnp.float32),
            pltpu.VMEM((K, D), jnp.float32),
            pltpu.SemaphoreType.DMA,
        ],
    )


def _tc_layer0_body(x_ref, w_ref, asrc_ref, adst_ref, h_ref, as_ref, ad_ref):
    h = jnp.dot(x_ref[...], w_ref[...], preferred_element_type=jnp.float32)
    h_ref[...] = h
    as_ref[...] = jnp.dot(h, asrc_ref[...], preferred_element_type=jnp.float32)
    ad_ref[...] = jnp.dot(h, adst_ref[...], preferred_element_type=jnp.float32)


def _tc_layern_body(p0_ref, p1_ref, d0_ref, d1_ref, b_ref, w_ref, asrc_ref,
                    adst_ref, h_ref, as_ref, ad_ref):
    agg = (p0_ref[...] + p1_ref[...]) / (d0_ref[...] + d1_ref[...] + 1e-16)
    hp = agg + b_ref[...]
    hp = jnp.where(hp > 0.0, hp, jnp.exp(hp) - 1.0)
    h = jnp.dot(hp, w_ref[...], preferred_element_type=jnp.float32)
    h_ref[...] = h
    as_ref[...] = jnp.dot(h, asrc_ref[...], preferred_element_type=jnp.float32)
    ad_ref[...] = jnp.dot(h, adst_ref[...], preferred_element_type=jnp.float32)


def _tc_final_body(p0_ref, p1_ref, d0_ref, d1_ref, b_ref, batch_ref,
                   fc1w_ref, fc1b_ref, fc2w_ref, fc2b_ref, out_ref,
                   pooled, counts):
    i = pl.program_id(0)
    agg = (p0_ref[...] + p1_ref[...]) / (d0_ref[...] + d1_ref[...] + 1e-16)
    hp = agg + b_ref[...]
    h = jnp.where(hp > 0.0, hp, jnp.exp(hp) - 1.0)

    gid = lax.broadcasted_iota(jnp.int32, (G, RBLK), 0)
    onehot = (batch_ref[...].reshape(1, RBLK) == gid).astype(jnp.float32)

    @pl.when(i == 0)
    def _():
        pooled[...] = jnp.zeros_like(pooled)
        counts[...] = jnp.zeros_like(counts)

    pooled[...] += jnp.dot(onehot, h, preferred_element_type=jnp.float32)
    counts[...] += jnp.sum(onehot, axis=1, keepdims=True)

    @pl.when(i == NBLK - 1)
    def _():
        pm = pooled[...] / jnp.maximum(counts[...], 1.0)
        z1 = jnp.dot(pm, fc1w_ref[...], preferred_element_type=jnp.float32)
        z1 = jnp.maximum(z1 + fc1b_ref[...], 0.0)
        z2 = jnp.dot(z1, fc2w_ref[...], preferred_element_type=jnp.float32)
        z2 = z2 + fc2b_ref[...]
        col = lax.broadcasted_iota(jnp.int32, (G, D), 1)
        zm = jnp.where(col < 2, z2, -1e30)
        m = jnp.max(zm, axis=1, keepdims=True)
        lse = jnp.log(jnp.sum(jnp.exp(zm - m), axis=1, keepdims=True)) + m
        out_ref[...] = z2 - lse


def _row_spec(shape):
    return pl.BlockSpec(shape, lambda i: (i, 0))


def _fixed_spec(shape):
    return pl.BlockSpec(shape, lambda i: (0, 0))


def _tc_layer0(x, W, a_src, a_dst):
    return pl.pallas_call(
        _tc_layer0_body,
        grid=(NBLK,),
        in_specs=[_row_spec((RBLK, D)), _fixed_spec((D, D)),
                  _fixed_spec((D, 1)), _fixed_spec((D, 1))],
        out_specs=[_row_spec((RBLK, D)), _row_spec((RBLK, 1)),
                   _row_spec((RBLK, 1))],
        out_shape=[jax.ShapeDtypeStruct((N, D), jnp.float32),
                   jax.ShapeDtypeStruct((N, 1), jnp.float32),
                   jax.ShapeDtypeStruct((N, 1), jnp.float32)],
    )(x, W, a_src.reshape(D, 1), a_dst.reshape(D, 1))


def _tc_layern(p0, p1, d0, d1, b, W, a_src, a_dst):
    return pl.pallas_call(
        _tc_layern_body,
        grid=(NBLK,),
        in_specs=[_row_spec((RBLK, D)), _row_spec((RBLK, D)),
                  _row_spec((RBLK, 1)), _row_spec((RBLK, 1)),
                  _fixed_spec((1, D)), _fixed_spec((D, D)),
                  _fixed_spec((D, 1)), _fixed_spec((D, 1))],
        out_specs=[_row_spec((RBLK, D)), _row_spec((RBLK, 1)),
                   _row_spec((RBLK, 1))],
        out_shape=[jax.ShapeDtypeStruct((N, D), jnp.float32),
                   jax.ShapeDtypeStruct((N, 1), jnp.float32),
                   jax.ShapeDtypeStruct((N, 1), jnp.float32)],
    )(p0, p1, d0, d1, b.reshape(1, D), W,
      a_src.reshape(D, 1), a_dst.reshape(D, 1))


def _tc_final(p0, p1, d0, d1, b, batch, fc1_w, fc1_b, fc2_w, fc2_b):
    fc2_wp = jnp.zeros((FC, D), jnp.float32).at[:, :2].set(fc2_w)
    fc2_bp = jnp.zeros((1, D), jnp.float32).at[0, :2].set(fc2_b)
    return pl.pallas_call(
        _tc_final_body,
        grid=(NBLK,),
        in_specs=[_row_spec((RBLK, D)), _row_spec((RBLK, D)),
                  _row_spec((RBLK, 1)), _row_spec((RBLK, 1)),
                  _fixed_spec((1, D)),
                  pl.BlockSpec((1, 1, RBLK), lambda i: (i, 0, 0)),
                  _fixed_spec((D, FC)), _fixed_spec((1, FC)),
                  _fixed_spec((FC, D)), _fixed_spec((1, D))],
        out_specs=_fixed_spec((G, D)),
        out_shape=jax.ShapeDtypeStruct((G, D), jnp.float32),
        scratch_shapes=[pltpu.VMEM((G, D), jnp.float32),
                        pltpu.VMEM((G, 1), jnp.float32)],
    )(p0, p1, d0, d1, b.reshape(1, D), batch.reshape(NBLK, 1, RBLK),
      fc1_w, fc1_b.reshape(1, FC), fc2_wp, fc2_bp)


def kernel(x, edge_index, batch, W0, att_src0, att_dst0, b0,
           W1, att_src1, att_dst1, b1, W2, att_src2, att_dst2, b2,
           fc1_w, fc1_b, fc2_w, fc2_b):
    e_total = edge_index.shape[1] + N
    n_chunks, per_worker = _num_chunks(e_total)
    padded = NW * per_worker

    loop = jnp.arange(N, dtype=jnp.int32)
    src = jnp.concatenate([
        edge_index[0].astype(jnp.int32), loop,
        jnp.zeros((padded - e_total,), jnp.int32)])
    dst = jnp.concatenate([
        edge_index[1].astype(jnp.int32), loop,
        jnp.full((padded - e_total,), N, jnp.int32)])

    sc_edge = _make_sc_edge(n_chunks, per_worker)

    h, a_s, a_d = _tc_layer0(x, W0, att_src0, att_dst0)
    for (b, W, asrc, adst) in ((b0, W1, att_src1, att_dst1),
                               (b1, W2, att_src2, att_dst2)):
        parts, dens = sc_edge(h, a_s.reshape(N), a_d.reshape(N), src, dst)
        h, a_s, a_d = _tc_layern(parts[0, :N], parts[1, :N],
                                 dens[0, :N].reshape(N, 1),
                                 dens[1, :N].reshape(N, 1),
                                 b, W, asrc, adst)
    parts, dens = sc_edge(h, a_s.reshape(N), a_d.reshape(N), src, dst)
    out = _tc_final(parts[0, :N], parts[1, :N],
                    dens[0, :N].reshape(N, 1), dens[1, :N].reshape(N, 1),
                    b2, batch.astype(jnp.int32), fc1_w, fc1_b, fc2_w, fc2_b)
    return out[:, :2]


# speedup vs baseline: 38.3396x; 1.5090x over previous
import functools
import jax
import jax.numpy as jnp
from jax import lax
from jax.experimental import pallas as pl
from jax.experimental.pallas import tpu as pltpu
from jax.experimental.pallas import tpu_sc as plsc

N = 10000
D = 128
G = 128
FC = 256
NC = 2
NS = 16
NW = NC * NS
K = 96
ROWS_PAD = 10240
TILE_ROWS = ROWS_PAD // NS
NBLK = 10
RBLK = N // NBLK


def _num_chunks(n_edges_total):
    n_chunks = -(-n_edges_total // (NW * K))
    n_chunks += (-n_chunks) % 4
    return n_chunks, n_chunks * K


def _sc_edge_body(h_hbm, as_hbm, ad_hbm, e_hbm, acc, den, asv, adv, iring,
                  pvs, rowss, gsems, ssems, psems, isems, n_chunks):
    cid = lax.axis_index("c")
    sid = lax.axis_index("s")
    wid = sid * NC + cid
    tbase = sid * TILE_ROWS

    zv = jnp.zeros((16,), jnp.float32)
    rows0, pv0 = rowss[0], pvs[0]

    def zrow(i, _):
        for f in range(8):
            rows0[i, pl.ds(f * 16, 16)] = zv
        return 0

    lax.fori_loop(0, K, zrow, 0)

    def zp(i, _):
        pv0[pl.ds(i * 16, 16)] = zv
        return 0

    lax.fori_loop(0, K // 16, zp, 0)

    def zout(r, _):
        pltpu.sync_copy(rows0.at[pl.ds(0, 64)],
                        acc.at[pl.ds(tbase + r * 64, 64)])
        pltpu.sync_copy(pv0.at[pl.ds(0, 64)],
                        den.at[pl.ds(tbase + r * 64, 64)])
        return 0

    lax.fori_loop(0, TILE_ROWS // 64, zout, 0)
    plsc.subcore_barrier()

    pltpu.sync_copy(as_hbm, asv)
    pltpu.sync_copy(ad_hbm, adv)

    pltpu.async_copy(e_hbm.at[wid, 0], iring.at[0], isems[0])
    pltpu.async_copy(e_hbm.at[wid, 1], iring.at[1], isems[1])
    pltpu.make_async_copy(e_hbm.at[wid, 0], iring.at[0], isems[0]).wait()
    pltpu.async_copy(h_hbm.at[iring.at[0, 0]], rowss[0], gsems[0])

    def quad(t, _):
        for b in range(4):
            c = 4 * t + b
            q = b % 2
            pv, rows = pvs[q], rowss[q]
            sidx = iring.at[b, 0]
            didx = iring.at[b, 1]

            @pl.when(c >= 2)
            def _():
                pltpu.make_async_copy(pv, den.at[didx], psems[q]).wait()

            @pl.when(c + 2 < n_chunks)
            def _():
                pltpu.async_copy(e_hbm.at[wid, c + 2],
                                 iring.at[(b + 2) % 4], isems[(b + 2) % 4])

            @pl.when(c >= 1)
            def _():
                pltpu.make_async_copy(e_hbm.at[wid, c],
                                      iring.at[b], isems[b]).wait()

            for g in range(K // 16):
                si = iring[b, 0, pl.ds(g * 16, 16)]
                di = iring[b, 1, pl.ds(g * 16, 16)]
                e = plsc.load_gather(asv, [si]) + plsc.load_gather(adv, [di])
                e = jnp.where(e > 0.0, e, 0.2 * e)
                pv[pl.ds(g * 16, 16)] = jnp.exp(e)
            pltpu.async_copy(pv, den.at[didx], psems[q], add=True)

            pltpu.make_async_copy(h_hbm.at[sidx], rows, gsems[q]).wait()

            @pl.when(c >= 1)
            def _():
                pltpu.make_async_copy(rowss[1 - q], acc.at[didx],
                                      ssems[1 - q]).wait()

            @pl.when(c + 1 < n_chunks)
            def _():
                pltpu.async_copy(h_hbm.at[iring.at[(b + 1) % 4, 0]],
                                 rowss[1 - q], gsems[1 - q])

            def scale(g, _):
                p16 = pv[pl.ds(g * 16, 16)]
                base16 = g * 16
                for j in range(16):
                    a = p16[j]
                    for f in range(8):
                        rows[base16 + j, pl.ds(f * 16, 16)] = (
                            rows[base16 + j, pl.ds(f * 16, 16)] * a)
                return 0

            lax.fori_loop(0, K // 16, scale, 0)
            pltpu.async_copy(rows, acc.at[didx], ssems[q], add=True)
        return 0

    lax.fori_loop(0, n_chunks // 4, quad, 0)
    pltpu.make_async_copy(rowss[1], acc.at[iring.at[0, 1]], ssems[1]).wait()
    pltpu.make_async_copy(pvs[0], den.at[iring.at[0, 1]], psems[0]).wait()
    pltpu.make_async_copy(pvs[1], den.at[iring.at[0, 1]], psems[1]).wait()
    plsc.subcore_barrier()


def _make_sc_edge(n_chunks):
    mesh = plsc.VectorSubcoreMesh(core_axis_name="c", subcore_axis_name="s",
                                  num_cores=NC, num_subcores=NS)

    def body(h_hbm, as_hbm, ad_hbm, e_hbm, outp_hbm, denp_hbm,
             acc, den, asv, adv, iring, pv0, pv1, rows0, rows1,
             gs0, gs1, ss0, ss1, ps0, ps1, is0, is1, is2, is3):
        cid = lax.axis_index("c")
        sid = lax.axis_index("s")
        _sc_edge_body(h_hbm, as_hbm, ad_hbm, e_hbm, acc, den, asv, adv,
                      iring, (pv0, pv1), (rows0, rows1), (gs0, gs1),
                      (ss0, ss1), (ps0, ps1), (is0, is1, is2, is3), n_chunks)
        tbase = sid * TILE_ROWS
        pltpu.sync_copy(acc.at[pl.ds(tbase, TILE_ROWS)],
                        outp_hbm.at[cid, pl.ds(tbase, TILE_ROWS)])
        pltpu.sync_copy(den.at[pl.ds(tbase, TILE_ROWS)],
                        denp_hbm.at[cid, pl.ds(tbase, TILE_ROWS)])

    return pl.kernel(
        body,
        out_type=[
            jax.ShapeDtypeStruct((NC, ROWS_PAD, D), jnp.float32),
            jax.ShapeDtypeStruct((NC, ROWS_PAD), jnp.float32),
        ],
        mesh=mesh,
        compiler_params=pltpu.CompilerParams(needs_layout_passes=False),
        scratch_types=[
            pltpu.VMEM_SHARED((ROWS_PAD, D), jnp.float32),
            pltpu.VMEM_SHARED((ROWS_PAD,), jnp.float32),
            pltpu.VMEM((N,), jnp.float32),
            pltpu.VMEM((N,), jnp.float32),
            pltpu.VMEM((4, 2, K), jnp.int32),
            pltpu.VMEM((K,), jnp.float32),
            pltpu.VMEM((K,), jnp.float32),
            pltpu.VMEM((K, D), jnp.float32),
            pltpu.VMEM((K, D), jnp.float32),
        ] + [pltpu.SemaphoreType.DMA] * 10,
    )


def _tc_layer0_body(x_ref, w_ref, asrc_ref, adst_ref, h_ref, as_ref, ad_ref):
    h = jnp.dot(x_ref[...], w_ref[...], preferred_element_type=jnp.float32)
    h_ref[...] = h
    as_ref[...] = jnp.dot(h, asrc_ref[...], preferred_element_type=jnp.float32)
    ad_ref[...] = jnp.dot(h, adst_ref[...], preferred_element_type=jnp.float32)


def _tc_layern_body(p0_ref, p1_ref, d0_ref, d1_ref, b_ref, w_ref, asrc_ref,
                    adst_ref, h_ref, as_ref, ad_ref):
    agg = (p0_ref[...] + p1_ref[...]) / (d0_ref[...] + d1_ref[...] + 1e-16)
    hp = agg + b_ref[...]
    hp = jnp.where(hp > 0.0, hp, jnp.exp(hp) - 1.0)
    h = jnp.dot(hp, w_ref[...], preferred_element_type=jnp.float32)
    h_ref[...] = h
    as_ref[...] = jnp.dot(h, asrc_ref[...], preferred_element_type=jnp.float32)
    ad_ref[...] = jnp.dot(h, adst_ref[...], preferred_element_type=jnp.float32)


def _tc_final_body(p0_ref, p1_ref, d0_ref, d1_ref, b_ref, batch_ref,
                   fc1w_ref, fc1b_ref, fc2w_ref, fc2b_ref, out_ref,
                   pooled, counts):
    i = pl.program_id(0)
    agg = (p0_ref[...] + p1_ref[...]) / (d0_ref[...] + d1_ref[...] + 1e-16)
    hp = agg + b_ref[...]
    h = jnp.where(hp > 0.0, hp, jnp.exp(hp) - 1.0)

    gid = lax.broadcasted_iota(jnp.int32, (G, RBLK), 0)
    onehot = (batch_ref[...].reshape(1, RBLK) == gid).astype(jnp.float32)

    @pl.when(i == 0)
    def _():
        pooled[...] = jnp.zeros_like(pooled)
        counts[...] = jnp.zeros_like(counts)

    pooled[...] += jnp.dot(onehot, h, preferred_element_type=jnp.float32)
    counts[...] += jnp.sum(onehot, axis=1, keepdims=True)

    @pl.when(i == NBLK - 1)
    def _():
        pm = pooled[...] / jnp.maximum(counts[...], 1.0)
        z1 = jnp.dot(pm, fc1w_ref[...], preferred_element_type=jnp.float32)
        z1 = jnp.maximum(z1 + fc1b_ref[...], 0.0)
        z2 = jnp.dot(z1, fc2w_ref[...], preferred_element_type=jnp.float32)
        z2 = z2 + fc2b_ref[...]
        col = lax.broadcasted_iota(jnp.int32, (G, D), 1)
        zm = jnp.where(col < 2, z2, -1e30)
        m = jnp.max(zm, axis=1, keepdims=True)
        lse = jnp.log(jnp.sum(jnp.exp(zm - m), axis=1, keepdims=True)) + m
        out_ref[...] = z2 - lse


def _row_spec(shape):
    return pl.BlockSpec(shape, lambda i: (i, 0))


def _fixed_spec(shape):
    return pl.BlockSpec(shape, lambda i: (0, 0))


def _tc_layer0(x, W, a_src, a_dst):
    return pl.pallas_call(
        _tc_layer0_body,
        grid=(NBLK,),
        in_specs=[_row_spec((RBLK, D)), _fixed_spec((D, D)),
                  _fixed_spec((D, 1)), _fixed_spec((D, 1))],
        out_specs=[_row_spec((RBLK, D)), _row_spec((RBLK, 1)),
                   _row_spec((RBLK, 1))],
        out_shape=[jax.ShapeDtypeStruct((N, D), jnp.float32),
                   jax.ShapeDtypeStruct((N, 1), jnp.float32),
                   jax.ShapeDtypeStruct((N, 1), jnp.float32)],
    )(x, W, a_src.reshape(D, 1), a_dst.reshape(D, 1))


def _tc_layern(p0, p1, d0, d1, b, W, a_src, a_dst):
    return pl.pallas_call(
        _tc_layern_body,
        grid=(NBLK,),
        in_specs=[_row_spec((RBLK, D)), _row_spec((RBLK, D)),
                  _row_spec((RBLK, 1)), _row_spec((RBLK, 1)),
                  _fixed_spec((1, D)), _fixed_spec((D, D)),
                  _fixed_spec((D, 1)), _fixed_spec((D, 1))],
        out_specs=[_row_spec((RBLK, D)), _row_spec((RBLK, 1)),
                   _row_spec((RBLK, 1))],
        out_shape=[jax.ShapeDtypeStruct((N, D), jnp.float32),
                   jax.ShapeDtypeStruct((N, 1), jnp.float32),
                   jax.ShapeDtypeStruct((N, 1), jnp.float32)],
    )(p0, p1, d0, d1, b.reshape(1, D), W,
      a_src.reshape(D, 1), a_dst.reshape(D, 1))


def _tc_final(p0, p1, d0, d1, b, batch, fc1_w, fc1_b, fc2_w, fc2_b):
    fc2_wp = jnp.zeros((FC, D), jnp.float32).at[:, :2].set(fc2_w)
    fc2_bp = jnp.zeros((1, D), jnp.float32).at[0, :2].set(fc2_b)
    return pl.pallas_call(
        _tc_final_body,
        grid=(NBLK,),
        in_specs=[_row_spec((RBLK, D)), _row_spec((RBLK, D)),
                  _row_spec((RBLK, 1)), _row_spec((RBLK, 1)),
                  _fixed_spec((1, D)),
                  pl.BlockSpec((1, 1, RBLK), lambda i: (i, 0, 0)),
                  _fixed_spec((D, FC)), _fixed_spec((1, FC)),
                  _fixed_spec((FC, D)), _fixed_spec((1, D))],
        out_specs=_fixed_spec((G, D)),
        out_shape=jax.ShapeDtypeStruct((G, D), jnp.float32),
        scratch_shapes=[pltpu.VMEM((G, D), jnp.float32),
                        pltpu.VMEM((G, 1), jnp.float32)],
    )(p0, p1, d0, d1, b.reshape(1, D), batch.reshape(NBLK, 1, RBLK),
      fc1_w, fc1_b.reshape(1, FC), fc2_wp, fc2_bp)


def kernel(x, edge_index, batch, W0, att_src0, att_dst0, b0,
           W1, att_src1, att_dst1, b1, W2, att_src2, att_dst2, b2,
           fc1_w, fc1_b, fc2_w, fc2_b):
    e_total = edge_index.shape[1] + N
    n_chunks, per_worker = _num_chunks(e_total)
    padded = NW * per_worker

    loop = jnp.arange(N, dtype=jnp.int32)
    src = jnp.concatenate([
        edge_index[0].astype(jnp.int32), loop,
        jnp.zeros((padded - e_total,), jnp.int32)]).reshape(NW, n_chunks, K)
    dst = jnp.concatenate([
        edge_index[1].astype(jnp.int32), loop,
        jnp.full((padded - e_total,), N, jnp.int32)]).reshape(NW, n_chunks, K)
    e_all = jnp.stack([src, dst], axis=2)

    sc_edge = _make_sc_edge(n_chunks)

    h, a_s, a_d = _tc_layer0(x, W0, att_src0, att_dst0)
    for (b, W, asrc, adst) in ((b0, W1, att_src1, att_dst1),
                               (b1, W2, att_src2, att_dst2)):
        parts, dens = sc_edge(h, a_s.reshape(N), a_d.reshape(N), e_all)
        h, a_s, a_d = _tc_layern(parts[0, :N], parts[1, :N],
                                 dens[0, :N].reshape(N, 1),
                                 dens[1, :N].reshape(N, 1),
                                 b, W, asrc, adst)
    parts, dens = sc_edge(h, a_s.reshape(N), a_d.reshape(N), e_all)
    out = _tc_final(parts[0, :N], parts[1, :N],
                    dens[0, :N].reshape(N, 1), dens[1, :N].reshape(N, 1),
                    b2, batch.astype(jnp.int32), fc1_w, fc1_b, fc2_w, fc2_b)
    return out[:, :2]


# speedup vs baseline: 39.6688x; 1.0347x over previous
import functools
import jax
import jax.numpy as jnp
from jax import lax
from jax.experimental import pallas as pl
from jax.experimental.pallas import tpu as pltpu
from jax.experimental.pallas import tpu_sc as plsc

N = 10000
D = 128
G = 128
FC = 256
NC = 2
NS = 16
NW = NC * NS
K = 96
ROWS_PAD = 10240
TILE_ROWS = ROWS_PAD // NS
NBLK = 10
RBLK = N // NBLK


def _num_chunks(n_edges_total):
    n_chunks = -(-n_edges_total // (NW * K))
    n_chunks += (-n_chunks) % 4
    return n_chunks, n_chunks * K


def _sc_edge_body(h_hbm, as_hbm, ad_hbm, e_hbm, acc, den, asv, adv, iring,
                  pvs, rowss, gsems, ssems, psems, isems, n_chunks):
    cid = lax.axis_index("c")
    sid = lax.axis_index("s")
    wid = sid * NC + cid
    tbase = sid * TILE_ROWS

    zv = jnp.zeros((16,), jnp.float32)
    rows0, pv0 = rowss[0], pvs[0]

    def zrow(i, _):
        for f in range(8):
            rows0[i, pl.ds(f * 16, 16)] = zv
        return 0

    lax.fori_loop(0, K, zrow, 0)

    def zp(i, _):
        pv0[pl.ds(i * 16, 16)] = zv
        return 0

    lax.fori_loop(0, K // 16, zp, 0)

    def zout(r, _):
        pltpu.sync_copy(rows0.at[pl.ds(0, 64)],
                        acc.at[pl.ds(tbase + r * 64, 64)])
        pltpu.sync_copy(pv0.at[pl.ds(0, 64)],
                        den.at[pl.ds(tbase + r * 64, 64)])
        return 0

    lax.fori_loop(0, TILE_ROWS // 64, zout, 0)
    plsc.subcore_barrier()

    pltpu.sync_copy(as_hbm, asv)
    pltpu.sync_copy(ad_hbm, adv)

    pltpu.async_copy(e_hbm.at[wid, 0], iring.at[0], isems[0])
    pltpu.async_copy(e_hbm.at[wid, 1], iring.at[1], isems[1])
    pltpu.make_async_copy(e_hbm.at[wid, 0], iring.at[0], isems[0]).wait()
    pltpu.async_copy(h_hbm.at[iring.at[0, 0]], rowss[0], gsems[0])

    def quad(t, _):
        for b in range(4):
            c = 4 * t + b
            q = b % 2
            pv, rows = pvs[q], rowss[q]
            sidx = iring.at[b, 0]
            didx = iring.at[b, 1]

            @pl.when(c >= 2)
            def _():
                pltpu.make_async_copy(pv, den.at[didx], psems[q]).wait()

            @pl.when(c + 2 < n_chunks)
            def _():
                pltpu.async_copy(e_hbm.at[wid, c + 2],
                                 iring.at[(b + 2) % 4], isems[(b + 2) % 4])

            @pl.when(c + 1 < n_chunks)
            def _():
                pltpu.make_async_copy(e_hbm.at[wid, c + 1],
                                      iring.at[(b + 1) % 4],
                                      isems[(b + 1) % 4]).wait()

            @pl.when(c >= 1)
            def _():
                pltpu.make_async_copy(rowss[1 - q], acc.at[didx],
                                      ssems[1 - q]).wait()

            @pl.when(c + 1 < n_chunks)
            def _():
                pltpu.async_copy(h_hbm.at[iring.at[(b + 1) % 4, 0]],
                                 rowss[1 - q], gsems[1 - q])

            for g in range(K // 16):
                si = iring[b, 0, pl.ds(g * 16, 16)]
                di = iring[b, 1, pl.ds(g * 16, 16)]
                e = plsc.load_gather(asv, [si]) + plsc.load_gather(adv, [di])
                e = jnp.where(e > 0.0, e, 0.2 * e)
                pv[pl.ds(g * 16, 16)] = jnp.exp(e)
            pltpu.async_copy(pv, den.at[didx], psems[q], add=True)

            pltpu.make_async_copy(h_hbm.at[sidx], rows, gsems[q]).wait()

            def scale(g, _):
                p16 = pv[pl.ds(g * 16, 16)]
                base16 = g * 16
                for j in range(16):
                    a = p16[j]
                    for f in range(8):
                        rows[base16 + j, pl.ds(f * 16, 16)] = (
                            rows[base16 + j, pl.ds(f * 16, 16)] * a)
                return 0

            lax.fori_loop(0, K // 16, scale, 0)
            pltpu.async_copy(rows, acc.at[didx], ssems[q], add=True)
        return 0

    lax.fori_loop(0, n_chunks // 4, quad, 0)
    pltpu.make_async_copy(rowss[1], acc.at[iring.at[0, 1]], ssems[1]).wait()
    pltpu.make_async_copy(pvs[0], den.at[iring.at[0, 1]], psems[0]).wait()
    pltpu.make_async_copy(pvs[1], den.at[iring.at[0, 1]], psems[1]).wait()
    plsc.subcore_barrier()


def _make_sc_edge(n_chunks):
    mesh = plsc.VectorSubcoreMesh(core_axis_name="c", subcore_axis_name="s",
                                  num_cores=NC, num_subcores=NS)

    def body(h_hbm, as_hbm, ad_hbm, e_hbm, outp_hbm, denp_hbm,
             acc, den, asv, adv, iring, pv0, pv1, rows0, rows1,
             gs0, gs1, ss0, ss1, ps0, ps1, is0, is1, is2, is3):
        cid = lax.axis_index("c")
        sid = lax.axis_index("s")
        _sc_edge_body(h_hbm, as_hbm, ad_hbm, e_hbm, acc, den, asv, adv,
                      iring, (pv0, pv1), (rows0, rows1), (gs0, gs1),
                      (ss0, ss1), (ps0, ps1), (is0, is1, is2, is3), n_chunks)
        tbase = sid * TILE_ROWS
        pltpu.sync_copy(acc.at[pl.ds(tbase, TILE_ROWS)],
                        outp_hbm.at[cid, pl.ds(tbase, TILE_ROWS)])
        pltpu.sync_copy(den.at[pl.ds(tbase, TILE_ROWS)],
                        denp_hbm.at[cid, pl.ds(tbase, TILE_ROWS)])

    return pl.kernel(
        body,
        out_type=[
            jax.ShapeDtypeStruct((NC, ROWS_PAD, D), jnp.float32),
            jax.ShapeDtypeStruct((NC, ROWS_PAD), jnp.float32),
        ],
        mesh=mesh,
        compiler_params=pltpu.CompilerParams(needs_layout_passes=False),
        scratch_types=[
            pltpu.VMEM_SHARED((ROWS_PAD, D), jnp.float32),
            pltpu.VMEM_SHARED((ROWS_PAD,), jnp.float32),
            pltpu.VMEM((N,), jnp.float32),
            pltpu.VMEM((N,), jnp.float32),
            pltpu.VMEM((4, 2, K), jnp.int32),
            pltpu.VMEM((K,), jnp.float32),
            pltpu.VMEM((K,), jnp.float32),
            pltpu.VMEM((K, D), jnp.float32),
            pltpu.VMEM((K, D), jnp.float32),
        ] + [pltpu.SemaphoreType.DMA] * 10,
    )


def _tc_layer0_body(x_ref, w_ref, asrc_ref, adst_ref, h_ref, as_ref, ad_ref):
    h = jnp.dot(x_ref[...], w_ref[...], preferred_element_type=jnp.float32)
    h_ref[...] = h
    as_ref[...] = jnp.dot(h, asrc_ref[...], preferred_element_type=jnp.float32)
    ad_ref[...] = jnp.dot(h, adst_ref[...], preferred_element_type=jnp.float32)


def _tc_layern_body(p0_ref, p1_ref, d0_ref, d1_ref, b_ref, w_ref, asrc_ref,
                    adst_ref, h_ref, as_ref, ad_ref):
    agg = (p0_ref[...] + p1_ref[...]) / (d0_ref[...] + d1_ref[...] + 1e-16)
    hp = agg + b_ref[...]
    hp = jnp.where(hp > 0.0, hp, jnp.exp(hp) - 1.0)
    h = jnp.dot(hp, w_ref[...], preferred_element_type=jnp.float32)
    h_ref[...] = h
    as_ref[...] = jnp.dot(h, asrc_ref[...], preferred_element_type=jnp.float32)
    ad_ref[...] = jnp.dot(h, adst_ref[...], preferred_element_type=jnp.float32)


def _tc_final_body(p0_ref, p1_ref, d0_ref, d1_ref, b_ref, batch_ref,
                   fc1w_ref, fc1b_ref, fc2w_ref, fc2b_ref, out_ref,
                   pooled, counts):
    i = pl.program_id(0)
    agg = (p0_ref[...] + p1_ref[...]) / (d0_ref[...] + d1_ref[...] + 1e-16)
    hp = agg + b_ref[...]
    h = jnp.where(hp > 0.0, hp, jnp.exp(hp) - 1.0)

    gid = lax.broadcasted_iota(jnp.int32, (G, RBLK), 0)
    onehot = (batch_ref[...].reshape(1, RBLK) == gid).astype(jnp.float32)

    @pl.when(i == 0)
    def _():
        pooled[...] = jnp.zeros_like(pooled)
        counts[...] = jnp.zeros_like(counts)

    pooled[...] += jnp.dot(onehot, h, preferred_element_type=jnp.float32)
    counts[...] += jnp.sum(onehot, axis=1, keepdims=True)

    @pl.when(i == NBLK - 1)
    def _():
        pm = pooled[...] / jnp.maximum(counts[...], 1.0)
        z1 = jnp.dot(pm, fc1w_ref[...], preferred_element_type=jnp.float32)
        z1 = jnp.maximum(z1 + fc1b_ref[...], 0.0)
        z2 = jnp.dot(z1, fc2w_ref[...], preferred_element_type=jnp.float32)
        z2 = z2 + fc2b_ref[...]
        col = lax.broadcasted_iota(jnp.int32, (G, D), 1)
        zm = jnp.where(col < 2, z2, -1e30)
        m = jnp.max(zm, axis=1, keepdims=True)
        lse = jnp.log(jnp.sum(jnp.exp(zm - m), axis=1, keepdims=True)) + m
        out_ref[...] = z2 - lse


def _row_spec(shape):
    return pl.BlockSpec(shape, lambda i: (i, 0))


def _fixed_spec(shape):
    return pl.BlockSpec(shape, lambda i: (0, 0))


def _tc_layer0(x, W, a_src, a_dst):
    return pl.pallas_call(
        _tc_layer0_body,
        grid=(NBLK,),
        in_specs=[_row_spec((RBLK, D)), _fixed_spec((D, D)),
                  _fixed_spec((D, 1)), _fixed_spec((D, 1))],
        out_specs=[_row_spec((RBLK, D)), _row_spec((RBLK, 1)),
                   _row_spec((RBLK, 1))],
        out_shape=[jax.ShapeDtypeStruct((N, D), jnp.float32),
                   jax.ShapeDtypeStruct((N, 1), jnp.float32),
                   jax.ShapeDtypeStruct((N, 1), jnp.float32)],
    )(x, W, a_src.reshape(D, 1), a_dst.reshape(D, 1))


def _tc_layern(p0, p1, d0, d1, b, W, a_src, a_dst):
    return pl.pallas_call(
        _tc_layern_body,
        grid=(NBLK,),
        in_specs=[_row_spec((RBLK, D)), _row_spec((RBLK, D)),
                  _row_spec((RBLK, 1)), _row_spec((RBLK, 1)),
                  _fixed_spec((1, D)), _fixed_spec((D, D)),
                  _fixed_spec((D, 1)), _fixed_spec((D, 1))],
        out_specs=[_row_spec((RBLK, D)), _row_spec((RBLK, 1)),
                   _row_spec((RBLK, 1))],
        out_shape=[jax.ShapeDtypeStruct((N, D), jnp.float32),
                   jax.ShapeDtypeStruct((N, 1), jnp.float32),
                   jax.ShapeDtypeStruct((N, 1), jnp.float32)],
    )(p0, p1, d0, d1, b.reshape(1, D), W,
      a_src.reshape(D, 1), a_dst.reshape(D, 1))


def _tc_final(p0, p1, d0, d1, b, batch, fc1_w, fc1_b, fc2_w, fc2_b):
    fc2_wp = jnp.zeros((FC, D), jnp.float32).at[:, :2].set(fc2_w)
    fc2_bp = jnp.zeros((1, D), jnp.float32).at[0, :2].set(fc2_b)
    return pl.pallas_call(
        _tc_final_body,
        grid=(NBLK,),
        in_specs=[_row_spec((RBLK, D)), _row_spec((RBLK, D)),
                  _row_spec((RBLK, 1)), _row_spec((RBLK, 1)),
                  _fixed_spec((1, D)),
                  pl.BlockSpec((1, 1, RBLK), lambda i: (i, 0, 0)),
                  _fixed_spec((D, FC)), _fixed_spec((1, FC)),
                  _fixed_spec((FC, D)), _fixed_spec((1, D))],
        out_specs=_fixed_spec((G, D)),
        out_shape=jax.ShapeDtypeStruct((G, D), jnp.float32),
        scratch_shapes=[pltpu.VMEM((G, D), jnp.float32),
                        pltpu.VMEM((G, 1), jnp.float32)],
    )(p0, p1, d0, d1, b.reshape(1, D), batch.reshape(NBLK, 1, RBLK),
      fc1_w, fc1_b.reshape(1, FC), fc2_wp, fc2_bp)


def kernel(x, edge_index, batch, W0, att_src0, att_dst0, b0,
           W1, att_src1, att_dst1, b1, W2, att_src2, att_dst2, b2,
           fc1_w, fc1_b, fc2_w, fc2_b):
    e_total = edge_index.shape[1] + N
    n_chunks, per_worker = _num_chunks(e_total)
    padded = NW * per_worker

    loop = jnp.arange(N, dtype=jnp.int32)
    src = jnp.concatenate([
        edge_index[0].astype(jnp.int32), loop,
        jnp.zeros((padded - e_total,), jnp.int32)]).reshape(NW, n_chunks, K)
    dst = jnp.concatenate([
        edge_index[1].astype(jnp.int32), loop,
        jnp.full((padded - e_total,), N, jnp.int32)]).reshape(NW, n_chunks, K)
    e_all = jnp.stack([src, dst], axis=2)

    sc_edge = _make_sc_edge(n_chunks)

    h, a_s, a_d = _tc_layer0(x, W0, att_src0, att_dst0)
    for (b, W, asrc, adst) in ((b0, W1, att_src1, att_dst1),
                               (b1, W2, att_src2, att_dst2)):
        parts, dens = sc_edge(h, a_s.reshape(N), a_d.reshape(N), e_all)
        h, a_s, a_d = _tc_layern(parts[0, :N], parts[1, :N],
                                 dens[0, :N].reshape(N, 1),
                                 dens[1, :N].reshape(N, 1),
                                 b, W, asrc, adst)
    parts, dens = sc_edge(h, a_s.reshape(N), a_d.reshape(N), e_all)
    out = _tc_final(parts[0, :N], parts[1, :N],
                    dens[0, :N].reshape(N, 1), dens[1, :N].reshape(N, 1),
                    b2, batch.astype(jnp.int32), fc1_w, fc1_b, fc2_w, fc2_b)
    return out[:, :2]
